# Initial kernel scaffold; baseline (speedup 1.0000x reference)
import jax
import jax.numpy as jnp
from jax.experimental import pallas as pl


def kernel(x, W):
    raise NotImplementedError("write your pallas kernel here")


# speedup vs baseline: 1.1029x; 1.1029x over previous
import functools

import jax
import jax.numpy as jnp
from jax import lax
from jax.experimental import pallas as pl
from jax.experimental.pallas import tpu as pltpu
from jax.experimental.pallas import tpu_sc as plsc

D = 32
NC = 2
NS = 16
NW = NC * NS
IDXW = 128
GROUP = 1024


def _make_kernel(B):
    b_per_w = B // NW
    n_rows = b_per_w // IDXW
    n_groups = b_per_w // GROUP
    g_rows = GROUP // IDXW

    mesh = plsc.VectorSubcoreMesh(core_axis_name="c", subcore_axis_name="s")

    @functools.partial(
        pl.kernel,
        out_type=jax.ShapeDtypeStruct((B, D), jnp.float32),
        mesh=mesh,
        scratch_types=[
            pltpu.VMEM((n_rows, IDXW), jnp.int32),
            pltpu.VMEM((GROUP, D), jnp.float32),
            pltpu.SemaphoreType.DMA,
        ],
        compiler_params=pltpu.CompilerParams(use_tc_tiling_on_sc=False),
    )
    def emb(x_hbm, w_hbm, out_hbm, idx_v, rows_v, sem):
        wid = lax.axis_index("s") * NC + lax.axis_index("c")
        base = wid * b_per_w
        pltpu.sync_copy(x_hbm.at[pl.ds(wid * n_rows, n_rows)], idx_v)

        def group_body(g, carry):
            copies = []
            for j in range(g_rows):
                copies.append(pltpu.async_copy(
                    w_hbm.at[idx_v.at[g * g_rows + j]],
                    rows_v.at[pl.ds(j * IDXW, IDXW)],
                    sem,
                ))
            for c in copies:
                c.wait()
            pltpu.sync_copy(rows_v, out_hbm.at[pl.ds(base + g * GROUP, GROUP)])
            return carry

        lax.fori_loop(0, n_groups, group_body, 0)

    return emb


def kernel(x, W):
    orig_shape = x.shape
    xf = jnp.reshape(x.astype(jnp.int32), (-1, IDXW))
    B = xf.shape[0] * IDXW
    assert B % (NW * GROUP) == 0
    out = _make_kernel(B)(xf, W)
    return jnp.reshape(out, orig_shape + (D,))


# speedup vs baseline: 1.1099x; 1.0064x over previous
import functools

import jax
import jax.numpy as jnp
from jax import lax
from jax.experimental import pallas as pl
from jax.experimental.pallas import tpu as pltpu
from jax.experimental.pallas import tpu_sc as plsc

D = 32
NC = 2
NS = 16
NW = NC * NS
IDXW = 128
GROUP = 1024


def _make_kernel(B):
    b_per_w = B // NW
    n_rows = b_per_w // IDXW
    n_groups = b_per_w // GROUP
    g_rows = GROUP // IDXW

    mesh = plsc.VectorSubcoreMesh(core_axis_name="c", subcore_axis_name="s")

    @functools.partial(
        pl.kernel,
        out_type=jax.ShapeDtypeStruct((B, D), jnp.float32),
        mesh=mesh,
        scratch_types=[
            pltpu.VMEM((n_rows, IDXW), jnp.int32),
            pltpu.VMEM((2, GROUP, D), jnp.float32),
            pltpu.SemaphoreType.DMA,
            pltpu.SemaphoreType.DMA,
        ],
        compiler_params=pltpu.CompilerParams(use_tc_tiling_on_sc=False),
    )
    def emb(x_hbm, w_hbm, out_hbm, idx_v, rows_v, sem_g, sem_o):
        wid = lax.axis_index("s") * NC + lax.axis_index("c")
        base = wid * b_per_w
        pltpu.sync_copy(x_hbm.at[pl.ds(wid * n_rows, n_rows)], idx_v)

        def gather_group(g, buf):
            copies = []
            for j in range(g_rows):
                copies.append(pltpu.async_copy(
                    w_hbm.at[idx_v.at[g * g_rows + j]],
                    rows_v.at[buf].at[pl.ds(j * IDXW, IDXW)],
                    sem_g,
                ))
            return copies

        def store_group(g, buf):
            return pltpu.async_copy(
                rows_v.at[buf], out_hbm.at[pl.ds(base + g * GROUP, GROUP)],
                sem_o,
            )

        for c in gather_group(0, 0):
            c.wait()
        store_group(0, 0)

        def group_body(g, carry):
            buf = g % 2
            for c in gather_group(g, buf):
                c.wait()
            pltpu.make_async_copy(
                rows_v.at[1 - buf], out_hbm.at[pl.ds(base, GROUP)], sem_o
            ).wait()
            store_group(g, buf)
            return carry

        lax.fori_loop(1, n_groups, group_body, 0)
        pltpu.make_async_copy(
            rows_v.at[0], out_hbm.at[pl.ds(base, GROUP)], sem_o
        ).wait()

    return emb


def kernel(x, W):
    orig_shape = x.shape
    xf = jnp.reshape(x.astype(jnp.int32), (-1, IDXW))
    B = xf.shape[0] * IDXW
    assert B % (NW * GROUP) == 0
    out = _make_kernel(B)(xf, W)
    return jnp.reshape(out, orig_shape + (D,))


# speedup vs baseline: 1.6487x; 1.4855x over previous
import functools

import jax
import jax.numpy as jnp
from jax import lax
from jax.experimental import pallas as pl
from jax.experimental.pallas import tpu as pltpu
from jax.experimental.pallas import tpu_sc as plsc

S = 50
NB = 16384
D = 32
NC = 2
NS = 16
NW = NC * NS
IDXW = 128
CPW = (S * NB // IDXW) // NW
G = 4
GROUP = G * IDXW
G_PER_W = CPW // G
ET = D // 8


def _emb_kernel(x_hbm, w_hbm, out_hbm, idx_v, in_v, out_v, sem_g, sem_o):
    wid = lax.axis_index("s") * NC + lax.axis_index("c")
    pltpu.sync_copy(x_hbm.at[pl.ds(wid * CPW, CPW)], idx_v)

    lane = lax.iota(jnp.int32, 16)

    def gather_group(g, p):
        for j in range(G):
            pltpu.async_copy(
                w_hbm.at[idx_v.at[g * G + j]],
                in_v.at[p].at[pl.ds(j * IDXW, IDXW)],
                sem_g,
            )

    def wait_gather(p):
        for j in range(G):
            pltpu.make_async_copy(
                w_hbm.at[pl.ds(0, IDXW)],
                in_v.at[p].at[pl.ds(j * IDXW, IDXW)],
                sem_g,
            ).wait()

    def transpose_group(p):
        in_ref = in_v.at[p]
        def t_body(t, carry):
            et = t // (G * 8)
            btc = (t // 8) % G
            ei = t % 8
            e = et * 8 + ei
            col = jnp.full((16,), e, jnp.int32)
            for big in range(8):
                rows = (btc * 128 + big * 16) + lane
                v = plsc.load_gather(in_ref, [rows, col])
                out_v[p, et, btc, ei, pl.ds(big * 16, 16)] = v
            return carry
        lax.fori_loop(0, ET * G * 8, t_body, 0)

    def store_group(g, p):
        c0 = wid * CPW + g * G
        s = c0 // 128
        bt0 = c0 % 128
        for et in range(ET):
            pltpu.async_copy(
                out_v.at[p].at[et],
                out_hbm.at[s, et, pl.ds(bt0, G)],
                sem_o,
            )

    def wait_store(p):
        for et in range(ET):
            pltpu.make_async_copy(
                out_v.at[p].at[et],
                out_hbm.at[0, et, pl.ds(0, G)],
                sem_o,
            ).wait()

    gather_group(0, 0)

    def group_body(g, carry):
        p = g % 2
        wait_gather(p)

        @pl.when(g + 1 < G_PER_W)
        def _():
            gather_group(g + 1, 1 - p)

        @pl.when(g >= 2)
        def _():
            wait_store(p)

        transpose_group(p)
        store_group(g, p)
        return carry

    lax.fori_loop(0, G_PER_W, group_body, 0)
    wait_store(0)
    wait_store(1)


def _make_kernel():
    m = plsc.VectorSubcoreMesh(core_axis_name="c", subcore_axis_name="s")
    return functools.partial(
        pl.kernel,
        out_type=jax.ShapeDtypeStruct((S, ET, 128, 8, 128), jnp.float32),
        mesh=m,
        scratch_types=[
            pltpu.VMEM((CPW, IDXW), jnp.int32),
            pltpu.VMEM((2, GROUP, D), jnp.float32),
            pltpu.VMEM((2, ET, G, 8, 128), jnp.float32),
            pltpu.SemaphoreType.DMA,
            pltpu.SemaphoreType.DMA,
        ],
        compiler_params=pltpu.CompilerParams(
            use_tc_tiling_on_sc=False, needs_layout_passes=False
        ),
    )(_emb_kernel)


def kernel(x, W):
    xt = jnp.reshape(jnp.transpose(x.astype(jnp.int32)), (S * NB // IDXW, IDXW))
    n = _make_kernel()(xt, W)
    return jnp.reshape(jnp.transpose(n, (2, 4, 0, 1, 3)), (NB, S, D))


# speedup vs baseline: 2.8051x; 1.7014x over previous
import functools

import jax
import jax.numpy as jnp
from jax import lax
from jax.experimental import pallas as pl
from jax.experimental.pallas import tpu as pltpu
from jax.experimental.pallas import tpu_sc as plsc

S = 50
NB = 16384
D = 32
NC = 2
NS = 16
NW = NC * NS
IDXW = 128
CPW = (S * NB // IDXW) // NW
G = 4
GROUP = G * IDXW
G_PER_W = CPW // G
ET = D // 8
PADW = 129


def _emb_kernel(x_hbm, w_hbm, out_hbm, idx_v, in_v, out_v, sem_g, sem_o):
    wid = lax.axis_index("s") * NC + lax.axis_index("c")
    pltpu.sync_copy(x_hbm.at[pl.ds(wid * CPW, CPW)], idx_v)

    lane = lax.iota(jnp.int32, 16)

    def gather_group(g, p):
        for j in range(G):
            pltpu.async_copy(
                w_hbm.at[idx_v.at[g * G + j]],
                in_v.at[p].at[pl.ds(j * IDXW, IDXW)],
                sem_g,
            )

    def wait_gather(p):
        for j in range(G):
            pltpu.make_async_copy(
                w_hbm.at[pl.ds(0, IDXW)],
                in_v.at[p].at[pl.ds(j * IDXW, IDXW)],
                sem_g,
            ).wait()

    et0 = lane // 8
    et1 = (lane + 16) // 8
    ei_c = lane % 8

    def transpose_group(p):
        out_ref = out_v.at[p]
        def btc_body(btc, carry):
            btc_s = jnp.full((16,), btc, jnp.int32)
            def q_body(qq, carry2):
                row = btc * IDXW + qq
                bi_s = jnp.full((16,), qq, jnp.int32)
                v0 = in_v[p, row, pl.ds(0, 16)]
                v1 = in_v[p, row, pl.ds(16, 16)]
                plsc.store_scatter(out_ref, [et0, btc_s, ei_c, bi_s], v0)
                plsc.store_scatter(out_ref, [et1, btc_s, ei_c, bi_s], v1)
                return carry2
            lax.fori_loop(0, IDXW, q_body, carry)
            return carry
        lax.fori_loop(0, G, btc_body, 0)

    def store_group(g, p):
        c0 = wid * CPW + g * G
        s = c0 // 128
        bt0 = c0 % 128
        for et in range(ET):
            pltpu.async_copy(
                out_v.at[p].at[et, pl.ds(0, G), pl.ds(0, 8), pl.ds(0, 128)],
                out_hbm.at[s, et, pl.ds(bt0, G)],
                sem_o,
            )

    def wait_store(p):
        for et in range(ET):
            pltpu.make_async_copy(
                out_v.at[p].at[et, pl.ds(0, G), pl.ds(0, 8), pl.ds(0, 128)],
                out_hbm.at[0, et, pl.ds(0, G)],
                sem_o,
            ).wait()

    gather_group(0, 0)

    def group_body(g, carry):
        p = g % 2
        wait_gather(p)

        @pl.when(g + 1 < G_PER_W)
        def _():
            gather_group(g + 1, 1 - p)

        @pl.when(g >= 2)
        def _():
            wait_store(p)

        transpose_group(p)
        store_group(g, p)
        return carry

    lax.fori_loop(0, G_PER_W, group_body, 0)
    wait_store(0)
    wait_store(1)


def _make_kernel():
    m = plsc.VectorSubcoreMesh(core_axis_name="c", subcore_axis_name="s")
    return functools.partial(
        pl.kernel,
        out_type=jax.ShapeDtypeStruct((S, ET, 128, 8, 128), jnp.float32),
        mesh=m,
        scratch_types=[
            pltpu.VMEM((CPW, IDXW), jnp.int32),
            pltpu.VMEM((2, GROUP, D), jnp.float32),
            pltpu.VMEM((2, ET, G, 8, PADW), jnp.float32),
            pltpu.SemaphoreType.DMA,
            pltpu.SemaphoreType.DMA,
        ],
        compiler_params=pltpu.CompilerParams(
            use_tc_tiling_on_sc=False, needs_layout_passes=False
        ),
    )(_emb_kernel)


def kernel(x, W):
    xt = jnp.reshape(jnp.transpose(x.astype(jnp.int32)), (S * NB // IDXW, IDXW))
    n = _make_kernel()(xt, W)
    return jnp.reshape(jnp.transpose(n, (2, 4, 0, 1, 3)), (NB, S, D))


# speedup vs baseline: 2.8552x; 1.0179x over previous
import functools

import jax
import jax.numpy as jnp
from jax import lax
from jax.experimental import pallas as pl
from jax.experimental.pallas import tpu as pltpu
from jax.experimental.pallas import tpu_sc as plsc

S = 50
NB = 16384
D = 32
NC = 2
NS = 16
NW = NC * NS
IDXW = 128
CPW = (S * NB // IDXW) // NW
G = 4
GROUP = G * IDXW
G_PER_W = CPW // G
ET = D // 8
PADW = 129
VOCABROWS = 1000000


def _emb_kernel(x_hbm, w_hbm, out_hbm, idx_v, in_v, out_v, sem_g, sem_o):
    wid = lax.axis_index("s") * NC + lax.axis_index("c")
    pltpu.sync_copy(x_hbm.at[pl.ds(wid * CPW, CPW)], idx_v)

    lane = lax.iota(jnp.int32, 16)

    def gather_group(g, p):
        for j in range(G):
            pltpu.async_copy(
                w_hbm.at[idx_v.at[g * G + j]],
                in_v.at[p].at[pl.ds(j * IDXW, IDXW)],
                sem_g,
            )

    def wait_gather(p):
        for j in range(G):
            pltpu.make_async_copy(
                w_hbm.at[pl.ds(0, IDXW)],
                in_v.at[p].at[pl.ds(j * IDXW, IDXW)],
                sem_g,
            ).wait()

    et0 = lane // 8
    et1 = (lane + 16) // 8
    ei_c = lane % 8

    def transpose_group(p):
        out_ref = out_v.at[p]
        def btc_body(btc, carry):
            btc_s = jnp.full((16,), btc, jnp.int32)
            def q_body(qq, carry2):
                row = btc * IDXW + qq
                bi_s = jnp.full((16,), qq, jnp.int32)
                v0 = in_v[p, row, pl.ds(0, 16)]
                v1 = in_v[p, row, pl.ds(16, 16)]
                plsc.store_scatter(out_ref, [et0, btc_s, ei_c, bi_s], v0)
                plsc.store_scatter(out_ref, [et1, btc_s, ei_c, bi_s], v1)
                return carry2
            lax.fori_loop(0, IDXW, q_body, carry)
            return carry
        lax.fori_loop(0, G, btc_body, 0)

    def store_group(g, p):
        c0 = wid * CPW + g * G
        s = c0 // 128
        bt0 = c0 % 128
        for et in range(ET):
            pltpu.async_copy(
                out_v.at[p].at[et, pl.ds(0, G), pl.ds(0, 8), pl.ds(0, 128)],
                out_hbm.at[s, et, pl.ds(bt0, G)],
                sem_o,
            )

    def wait_store(p):
        for et in range(ET):
            pltpu.make_async_copy(
                out_v.at[p].at[et, pl.ds(0, G), pl.ds(0, 8), pl.ds(0, 128)],
                out_hbm.at[0, et, pl.ds(0, G)],
                sem_o,
            ).wait()

    gather_group(0, 0)

    def group_body(g, carry):
        p = g % 2
        wait_gather(p)

        @pl.when(g + 1 < G_PER_W)
        def _():
            gather_group(g + 1, 1 - p)

        @pl.when(g >= 2)
        def _():
            wait_store(p)

        transpose_group(p)
        store_group(g, p)
        return carry

    lax.fori_loop(0, G_PER_W, group_body, 0)
    wait_store(0)
    wait_store(1)


def _make_kernel():
    m = plsc.VectorSubcoreMesh(core_axis_name="c", subcore_axis_name="s")
    return functools.partial(
        pl.kernel,
        out_type=jax.ShapeDtypeStruct((S, ET, 128, 8, 128), jnp.float32),
        mesh=m,
        scratch_types=[
            pltpu.VMEM((CPW, IDXW), jnp.int32),
            pltpu.VMEM((2, GROUP, D), jnp.float32),
            pltpu.VMEM((2, ET, G, 8, PADW), jnp.float32),
            pltpu.SemaphoreType.DMA,
            pltpu.SemaphoreType.DMA,
        ],
        compiler_params=pltpu.CompilerParams(
            use_tc_tiling_on_sc=False, needs_layout_passes=False
        ),
    )(_emb_kernel)


def kernel(x, W):
    xt = jnp.reshape(
        jnp.transpose(x.astype(jnp.int32) * 4), (S * NB // IDXW, IDXW)
    )
    w4m = jnp.reshape(
        jnp.pad(W, ((0, 0), (0, 128 - D))), (4 * VOCABROWS, D)
    )
    n = _make_kernel()(xt, w4m)
    return jnp.reshape(jnp.transpose(n, (2, 4, 0, 1, 3)), (NB, S, D))


# speedup vs baseline: 2.8865x; 1.0110x over previous
import functools

import jax
import jax.numpy as jnp
from jax import lax
from jax.experimental import pallas as pl
from jax.experimental.pallas import tpu as pltpu
from jax.experimental.pallas import tpu_sc as plsc

S = 50
NB = 16384
D = 32
NC = 2
NS = 16
NW = NC * NS
IDXW = 128
CPW = (S * NB // IDXW) // NW
G = 4
GROUP = G * IDXW
G_PER_W = CPW // G
ET = D // 8
PADW = 129
VOCABROWS = 1000000


def _emb_kernel(x_hbm, w_hbm, out_hbm, idx_v, in_v, out_v, sem_g, sem_o):
    wid = lax.axis_index("s") * NC + lax.axis_index("c")
    pltpu.sync_copy(x_hbm.at[pl.ds(wid * CPW, CPW)], idx_v)

    lane = lax.iota(jnp.int32, 16)

    def gather_group(g, p):
        for j in range(G):
            pltpu.async_copy(
                w_hbm.at[idx_v.at[g * G + j]],
                in_v.at[p].at[pl.ds(j * IDXW, IDXW)],
                sem_g,
            )

    def wait_gather(p):
        for j in range(G):
            pltpu.make_async_copy(
                w_hbm.at[pl.ds(0, IDXW)],
                in_v.at[p].at[pl.ds(j * IDXW, IDXW)],
                sem_g,
            ).wait()

    et0 = lane // 8
    et1 = (lane + 16) // 8
    ei_c = lane % 8

    def transpose_group(p):
        out_ref = out_v.at[p]
        def btc_body(btc, carry):
            btc_s = jnp.full((16,), btc, jnp.int32)
            def q_body(q4, carry2):
                for u in range(4):
                    qq = q4 * 4 + u
                    row = btc * IDXW + qq
                    bi_s = jnp.full((16,), qq, jnp.int32)
                    v0 = in_v[p, row, pl.ds(0, 16)]
                    v1 = in_v[p, row, pl.ds(16, 16)]
                    plsc.store_scatter(out_ref, [et0, btc_s, ei_c, bi_s], v0)
                    plsc.store_scatter(out_ref, [et1, btc_s, ei_c, bi_s], v1)
                return carry2
            lax.fori_loop(0, IDXW // 4, q_body, carry)
            return carry
        lax.fori_loop(0, G, btc_body, 0)

    def store_group(g, p):
        c0 = wid * CPW + g * G
        s = c0 // 128
        bt0 = c0 % 128
        for et in range(ET):
            pltpu.async_copy(
                out_v.at[p].at[et, pl.ds(0, G), pl.ds(0, 8), pl.ds(0, 128)],
                out_hbm.at[s, et, pl.ds(bt0, G)],
                sem_o,
            )

    def wait_store(p):
        for et in range(ET):
            pltpu.make_async_copy(
                out_v.at[p].at[et, pl.ds(0, G), pl.ds(0, 8), pl.ds(0, 128)],
                out_hbm.at[0, et, pl.ds(0, G)],
                sem_o,
            ).wait()

    gather_group(0, 0)

    def group_body(g, carry):
        p = g % 2
        wait_gather(p)

        @pl.when(g + 1 < G_PER_W)
        def _():
            gather_group(g + 1, 1 - p)

        @pl.when(g >= 2)
        def _():
            wait_store(p)

        transpose_group(p)
        store_group(g, p)
        return carry

    lax.fori_loop(0, G_PER_W, group_body, 0)
    wait_store(0)
    wait_store(1)


def _make_kernel():
    m = plsc.VectorSubcoreMesh(core_axis_name="c", subcore_axis_name="s")
    return functools.partial(
        pl.kernel,
        out_type=jax.ShapeDtypeStruct((S, ET, 128, 8, 128), jnp.float32),
        mesh=m,
        scratch_types=[
            pltpu.VMEM((CPW, IDXW), jnp.int32),
            pltpu.VMEM((2, GROUP, D), jnp.float32),
            pltpu.VMEM((2, ET, G, 8, PADW), jnp.float32),
            pltpu.SemaphoreType.DMA,
            pltpu.SemaphoreType.DMA,
        ],
        compiler_params=pltpu.CompilerParams(
            use_tc_tiling_on_sc=False, needs_layout_passes=False
        ),
    )(_emb_kernel)


def kernel(x, W):
    xt = jnp.reshape(
        jnp.transpose(x.astype(jnp.int32) * 4), (S * NB // IDXW, IDXW)
    )
    w4m = jnp.reshape(
        jnp.pad(W, ((0, 0), (0, 128 - D))), (4 * VOCABROWS, D)
    )
    n = _make_kernel()(xt, w4m)
    return jnp.reshape(jnp.transpose(n, (2, 4, 0, 1, 3)), (NB, S, D))
